# baseline (speedup 1.0000x reference)
import functools
import math

import jax
import jax.numpy as jnp
from jax import lax
from jax.experimental import pallas as pl
from jax.experimental.pallas import tpu as pltpu
from jax.experimental.pallas import tpu_sc as plsc

D_MODEL = 64
SCALE = 8.0
NUM_WORKERS = 32
CHUNK = 128
LANES = 16


@functools.partial(jax.jit, static_argnames=())
def _gather_scale(idx, lut):
    (b_total,) = idx.shape
    b_per_w = b_total // NUM_WORKERS
    steps = b_per_w // CHUNK

    mesh = plsc.VectorSubcoreMesh(core_axis_name="c", subcore_axis_name="s")

    @functools.partial(
        pl.kernel,
        mesh=mesh,
        out_type=jax.ShapeDtypeStruct((b_total, D_MODEL), jnp.float32),
        scratch_types=[
            pltpu.VMEM((b_per_w,), jnp.int32),
            pltpu.VMEM((CHUNK, D_MODEL), jnp.float32),
            pltpu.SemaphoreType.DMA,
        ],
        compiler_params=pltpu.CompilerParams(use_tc_tiling_on_sc=False),
    )
    def k(lut_hbm, idx_hbm, out_hbm, idx_v, rows_v, sem):
        wid = lax.axis_index("s") * 2 + lax.axis_index("c")
        base = wid * b_per_w
        pltpu.sync_copy(idx_hbm.at[pl.ds(base, b_per_w)], idx_v)

        def step(i, carry):
            c0 = i * CHUNK
            pltpu.async_copy(
                lut_hbm.at[idx_v.at[pl.ds(c0, CHUNK)]], rows_v, sem
            ).wait()

            def scale_row(r, carry2):
                for j in range(D_MODEL // LANES):
                    sl = pl.ds(j * LANES, LANES)
                    rows_v[r, sl] = rows_v[r, sl] * SCALE
                return carry2

            lax.fori_loop(0, CHUNK, scale_row, 0)
            pltpu.sync_copy(rows_v, out_hbm.at[pl.ds(base + c0, CHUNK)])
            return carry

        lax.fori_loop(0, steps, step, 0)

    return k(lut, idx)


def kernel(x, lut):
    b, h = x.shape
    idx = x.reshape(b * h)
    out = _gather_scale(idx, lut)
    return out.reshape(b, h, D_MODEL)


# speedup vs baseline: 1.2037x; 1.2037x over previous
import functools

import jax
import jax.numpy as jnp
from jax import lax
from jax.experimental import pallas as pl
from jax.experimental.pallas import tpu as pltpu
from jax.experimental.pallas import tpu_sc as plsc

D_MODEL = 64
SCALE = 8.0
NUM_WORKERS = 32
CHUNK = 128
LANES = 16
NBUF = 4


def _gather_scale(idx, lut):
    (b_total,) = idx.shape
    b_per_w = b_total // NUM_WORKERS
    steps = b_per_w // CHUNK
    outer = steps // NBUF

    mesh = plsc.VectorSubcoreMesh(core_axis_name="c", subcore_axis_name="s")

    @functools.partial(
        pl.kernel,
        mesh=mesh,
        out_type=jax.ShapeDtypeStruct((b_total, D_MODEL), jnp.float32),
        scratch_types=[
            pltpu.VMEM((b_per_w,), jnp.int32),
            pltpu.VMEM((NBUF, CHUNK, D_MODEL), jnp.float32),
            pltpu.VMEM((NBUF, CHUNK, D_MODEL), jnp.float32),
            [pltpu.SemaphoreType.DMA] * NBUF,
            [pltpu.SemaphoreType.DMA] * NBUF,
        ],
        compiler_params=pltpu.CompilerParams(use_tc_tiling_on_sc=False),
    )
    def k(lut_hbm, idx_hbm, out_hbm, idx_v, gbufs, obufs, gsems, wsems):
        wid = lax.axis_index("s") * 2 + lax.axis_index("c")
        base = wid * b_per_w
        pltpu.sync_copy(idx_hbm.at[pl.ds(base, b_per_w)], idx_v)

        for b in range(NBUF):
            pltpu.async_copy(
                lut_hbm.at[idx_v.at[pl.ds(b * CHUNK, CHUNK)]],
                gbufs.at[b],
                gsems[b],
            )

        def outer_body(g, carry):
            for b in range(NBUF):
                i = g * NBUF + b
                gb = gbufs.at[b]
                ob = obufs.at[b]
                pltpu.make_async_copy(
                    lut_hbm.at[idx_v.at[pl.ds(0, CHUNK)]], gb, gsems[b]
                ).wait()

                @pl.when(g > 0)
                def _():
                    pltpu.make_async_copy(
                        ob, out_hbm.at[pl.ds(0, CHUNK)], wsems[b]
                    ).wait()

                def scale_body(t, c):
                    r0 = t * 4
                    for rr in range(4):
                        for j in range(D_MODEL // LANES):
                            sl = pl.ds(j * LANES, LANES)
                            ob[r0 + rr, sl] = gb[r0 + rr, sl] * SCALE
                    return c

                lax.fori_loop(0, CHUNK // 4, scale_body, 0)

                @pl.when(g < outer - 1)
                def _():
                    pltpu.async_copy(
                        lut_hbm.at[idx_v.at[pl.ds((i + NBUF) * CHUNK, CHUNK)]],
                        gb,
                        gsems[b],
                    )

                pltpu.async_copy(
                    ob, out_hbm.at[pl.ds(base + i * CHUNK, CHUNK)], wsems[b]
                )
            return carry

        lax.fori_loop(0, outer, outer_body, 0)

        for b in range(NBUF):
            pltpu.make_async_copy(
                obufs.at[b], out_hbm.at[pl.ds(0, CHUNK)], wsems[b]
            ).wait()

    return k(lut, idx)


def kernel(x, lut):
    b, h = x.shape
    idx = x.reshape(b * h)
    out = _gather_scale(idx, lut)
    return out.reshape(b, h, D_MODEL)
